# Initial kernel scaffold; baseline (speedup 1.0000x reference)
import jax
import jax.numpy as jnp
from jax.experimental import pallas as pl


def kernel(features):
    raise NotImplementedError("write your pallas kernel here")


# speedup vs baseline: 7.2569x; 7.2569x over previous
import numpy as np
import jax
import jax.numpy as jnp
from jax import lax
from jax.experimental import pallas as pl
from jax.experimental.pallas import tpu as pltpu

_SCALES = (8, 16, 32)
_KPS = 4
_KNN = 20
_NPI = 50
_B = 2
_HW = 32
_D = 768


def _pool_matrix(s):
    bs = _HW // s
    P = np.zeros((s * s, _HW * _HW), np.float32)
    for i in range(s):
        for j in range(s):
            for y in range(i * bs, (i + 1) * bs):
                for x in range(j * bs, (j + 1) * bs):
                    P[i * s + j, y * _HW + x] = 1.0 / (bs * bs)
    return jnp.asarray(P)


def _rng_consts():
    key = jax.random.key(42)
    consts = []
    for s in _SCALES:
        n = s * s
        vinit = np.zeros((_B, _KPS - 1, n), np.float32)
        gumb = np.zeros((_B, _KPS - 1, n), np.float32)
        c0 = np.zeros((_B,), np.int32)
        for b in range(_B):
            for i in range(_KPS - 1):
                key, sk = jax.random.split(key)
                vinit[b, i] = np.asarray(jax.random.normal(sk, (n,), jnp.float32))
            key, sk = jax.random.split(key)
            c0[b] = int(jax.random.randint(sk, (), 0, n))
            for j in range(_KPS - 1):
                key, sk = jax.random.split(key)
                gumb[b, j] = np.asarray(jax.random.gumbel(sk, (n,), jnp.float32))
        consts.append((jnp.asarray(vinit), jnp.asarray(gumb), jnp.asarray(c0)))
    return consts


_POOL = {s: _pool_matrix(s) for s in _SCALES if s != _HW}
_RNG = _rng_consts()

_F32 = jnp.float32
_DN_1_1 = (((1,), (1,)), ((), ()))
_DN_1_0 = (((1,), (0,)), ((), ()))


def _mm(a, b, dn):
    return lax.dot_general(a, b, dn, preferred_element_type=_F32)


def _make_graph_kernel(s):
    n = s * s

    def body(x_ref, pool_ref, vinit_ref, gumb_ref, c0_ref, out_ref):
        b = pl.program_id(0)
        x = x_ref[...]
        if s == _HW:
            f = x
        else:
            f = _mm(pool_ref[...], x, _DN_1_0)

        ones_d = jnp.ones((1, _D), _F32)
        fsq = f * f
        fn_col = _mm(fsq, ones_d, _DN_1_1)
        fn_row = _mm(ones_d, fsq, _DN_1_1)
        G = _mm(f, f, _DN_1_1)
        dmat = jnp.sqrt(jnp.maximum(fn_col + fn_row - 2.0 * G, 0.0))

        big = _F32(np.inf)

        def min_body(_, m):
            return jnp.min(jnp.where(dmat > m, dmat, big), axis=1, keepdims=True)

        thr = lax.fori_loop(0, _KNN, min_body,
                            jnp.min(dmat, axis=1, keepdims=True))

        ri = lax.broadcasted_iota(jnp.int32, (n, n), 0)
        ci = lax.broadcasted_iota(jnp.int32, (n, n), 1)
        eye_b = ri == ci
        sel = (dmat <= thr) & jnp.logical_not(eye_b)
        A = jnp.where(sel, jnp.exp(dmat * -0.5), 0.0)
        eye_f = jnp.where(eye_b, _F32(1.0), _F32(0.0))
        At = _mm(A, eye_f, (((0,), (0,)), ((), ())))
        Wm = 0.5 * (A + At)

        ones_n = jnp.ones((1, n), _F32)
        Dv = _mm(ones_n, Wm, _DN_1_1)
        Dis = lax.rsqrt(Dv + 1e-8)

        def matvec(u):
            return _mm(u, Wm, _DN_1_1)

        mean_f = _mm(ones_d, f, _DN_1_1) * _F32(1.0 / _D)
        evs = []
        for i in range(_KPS):
            v0 = mean_f if i == 0 else vinit_ref[i - 1:i, :]
            v = v0 / (jnp.sqrt(jnp.sum(v0 * v0)) + 1e-8)

            def it_body(_, v, evs=tuple(evs)):
                vn = matvec(v * Dis) * Dis
                for ev in evs:
                    vn = vn - jnp.sum(vn * ev) * ev
                return vn / (jnp.sqrt(jnp.sum(vn * vn)) + 1e-8)

            evs.append(lax.fori_loop(0, _NPI, it_body, v))

        fnorm = f / jnp.maximum(jnp.sqrt(fn_col), 1e-12)
        E = jnp.concatenate(evs, axis=0)
        il = lax.broadcasted_iota(jnp.int32, (1, n), 1)

        c_idx = c0_ref[b]
        min_d = None
        for j in range(_KPS):
            oh = jnp.where(il == c_idx, _F32(1.0), _F32(0.0))
            out_ref[j:j + 1, :] = _mm(oh, f, _DN_1_0)
            if j == _KPS - 1:
                break
            fcn = _mm(oh, fnorm, _DN_1_0)
            diff = fnorm - fcn
            d2fn = _mm(ones_d, diff * diff, _DN_1_1)
            Ec = _mm(E, oh, _DN_1_1)
            dE = 2.0 * E - 2.0 * Ec
            d2E = jnp.sum(dE * dE, axis=0, keepdims=True)
            dcur = jnp.sqrt(jnp.maximum(d2E + d2fn, 0.0))
            min_d = dcur if min_d is None else jnp.minimum(min_d, dcur)
            probs = min_d * min_d
            probs = probs / (jnp.sum(probs) + 1e-8)
            score = jnp.log(probs + 1e-20) + gumb_ref[j:j + 1, :]
            mx = jnp.max(score)
            c_idx = jnp.min(jnp.where(score == mx, il, n))

    return body


def _scale_call(s, X):
    n = s * s
    si = _SCALES.index(s)
    vinit, gumb, c0 = _RNG[si]
    P = _POOL.get(s)
    if P is None:
        P = jnp.zeros((8, 128), _F32)
    return pl.pallas_call(
        _make_graph_kernel(s),
        grid=(_B,),
        in_specs=[
            pl.BlockSpec((None, _HW * _HW, _D), lambda b: (b, 0, 0)),
            pl.BlockSpec(P.shape, lambda b: (0, 0)),
            pl.BlockSpec((None, _KPS - 1, n), lambda b: (b, 0, 0)),
            pl.BlockSpec((None, _KPS - 1, n), lambda b: (b, 0, 0)),
            pl.BlockSpec(memory_space=pltpu.SMEM),
        ],
        out_specs=pl.BlockSpec((None, _KPS, _D), lambda b: (b, 0, 0)),
        out_shape=jax.ShapeDtypeStruct((_B, _KPS, _D), _F32),
    )(X, P, vinit, gumb, c0)


def kernel(features):
    X = features.reshape(_B, _HW * _HW, _D)
    outs = [_scale_call(s, X) for s in _SCALES]
    return jnp.concatenate(outs, axis=1)


# speedup vs baseline: 14.5305x; 2.0023x over previous
import numpy as np
import jax
import jax.numpy as jnp
from jax import lax
from jax.experimental import pallas as pl
from jax.experimental.pallas import tpu as pltpu

_SCALES = (8, 16, 32)
_KPS = 4
_KNN = 20
_NPI = 50
_B = 2
_HW = 32
_D = 768
_NG = len(_SCALES) * _B


def _pool_matrix(s):
    bs = _HW // s
    P = np.zeros((s * s, _HW * _HW), np.float32)
    for i in range(s):
        for j in range(s):
            for y in range(i * bs, (i + 1) * bs):
                for x in range(j * bs, (j + 1) * bs):
                    P[i * s + j, y * _HW + x] = 1.0 / (bs * bs)
    return jnp.asarray(P)


def _rng_consts():
    key = jax.random.key(42)
    vinits, gumbs, c0s = [], [], []
    for s in _SCALES:
        n = s * s
        vinit = np.zeros((_B, _KPS - 1, n), np.float32)
        gumb = np.zeros((_B, _KPS - 1, n), np.float32)
        for b in range(_B):
            for i in range(_KPS - 1):
                key, sk = jax.random.split(key)
                vinit[b, i] = np.asarray(jax.random.normal(sk, (n,), jnp.float32))
            key, sk = jax.random.split(key)
            c0s.append(int(jax.random.randint(sk, (), 0, n)))
            for j in range(_KPS - 1):
                key, sk = jax.random.split(key)
                gumb[b, j] = np.asarray(jax.random.gumbel(sk, (n,), jnp.float32))
        vinits.append(jnp.asarray(vinit))
        gumbs.append(jnp.asarray(gumb))
    return vinits, gumbs, jnp.asarray(np.asarray(c0s, np.int32))


_POOL = {s: _pool_matrix(s) for s in _SCALES if s != _HW}
_VINITS, _GUMBS, _C0 = _rng_consts()

_F32 = jnp.float32
_DN_1_1 = (((1,), (1,)), ((), ()))
_DN_1_0 = (((1,), (0,)), ((), ()))


def _mm(a, b, dn):
    return lax.dot_general(a, b, dn, preferred_element_type=_F32)


def _mega_body(x_ref, p8_ref, p16_ref, v8_ref, v16_ref, v32_ref,
               g8_ref, g16_ref, g32_ref, c0_ref, out_ref):
    ones_d = jnp.ones((1, _D), _F32)
    p_refs = {8: p8_ref, 16: p16_ref}
    v_refs = {8: v8_ref, 16: v16_ref, 32: v32_ref}
    g_refs = {8: g8_ref, 16: g16_ref, 32: g32_ref}

    fs, fn_cols, dmats = [], [], []
    for si, s in enumerate(_SCALES):
        n = s * s
        for b in range(_B):
            x = x_ref[b]
            f = x if s == _HW else _mm(p_refs[s][...], x, _DN_1_0)
            fsq = f * f
            fn_col = _mm(fsq, ones_d, _DN_1_1)
            fn_row = _mm(ones_d, fsq, _DN_1_1)
            G = _mm(f, f, _DN_1_1)
            dmat = jnp.sqrt(jnp.maximum(fn_col + fn_row - 2.0 * G, 0.0))
            fs.append(f)
            fn_cols.append(fn_col)
            dmats.append(dmat)

    big = _F32(np.inf)

    def thr_body(_, ms):
        return tuple(
            jnp.min(jnp.where(d > m, d, big), axis=1, keepdims=True)
            for d, m in zip(dmats, ms))

    thrs = lax.fori_loop(
        0, _KNN, thr_body,
        tuple(jnp.min(d, axis=1, keepdims=True) for d in dmats))

    Ms = []
    for g in range(_NG):
        n = dmats[g].shape[0]
        ri = lax.broadcasted_iota(jnp.int32, (n, n), 0)
        ci = lax.broadcasted_iota(jnp.int32, (n, n), 1)
        eye_b = ri == ci
        sel = (dmats[g] <= thrs[g]) & jnp.logical_not(eye_b)
        A = jnp.where(sel, jnp.exp(dmats[g] * -0.5), 0.0)
        eye_f = jnp.where(eye_b, _F32(1.0), _F32(0.0))
        At = _mm(A, eye_f, (((0,), (0,)), ((), ())))
        Wm = 0.5 * (A + At)
        ones_n = jnp.ones((1, n), _F32)
        Dv = _mm(ones_n, Wm, _DN_1_1)
        Dis = lax.rsqrt(Dv + 1e-8)
        Dis_col = _mm(Dis, eye_f, _DN_1_1)
        Ms.append(Wm * Dis * Dis_col)

    def norm1(v):
        return v / (jnp.sqrt(jnp.sum(v * v)) + 1e-8)

    evs = [[] for _ in range(_NG)]
    for i in range(_KPS):
        v0s = []
        for g in range(_NG):
            si, b = divmod(g, _B)
            if i == 0:
                v0 = _mm(ones_d, fs[g], _DN_1_1) * _F32(1.0 / _D)
            else:
                v0 = v_refs[_SCALES[si]][b, i - 1:i, :]
            v0s.append(norm1(v0))

        def pbody(_, vs, evs_t=tuple(tuple(e) for e in evs)):
            out = []
            for g in range(_NG):
                vn = _mm(vs[g], Ms[g], _DN_1_1)
                for ev in evs_t[g]:
                    vn = vn - jnp.sum(vn * ev) * ev
                out.append(norm1(vn))
            return tuple(out)

        vs = lax.fori_loop(0, _NPI, pbody, tuple(v0s))
        for g in range(_NG):
            evs[g].append(vs[g])

    fnorms, Es, ils, c_idx, min_d = [], [], [], [], [None] * _NG
    for g in range(_NG):
        n = fs[g].shape[0]
        fnorms.append(fs[g] / jnp.maximum(jnp.sqrt(fn_cols[g]), 1e-12))
        Es.append(jnp.concatenate(evs[g], axis=0))
        ils.append(lax.broadcasted_iota(jnp.int32, (1, n), 1))
        c_idx.append(c0_ref[g])

    for j in range(_KPS):
        for g in range(_NG):
            si, b = divmod(g, _B)
            n = fs[g].shape[0]
            il = ils[g]
            oh = jnp.where(il == c_idx[g], _F32(1.0), _F32(0.0))
            row = si * _KPS + j
            out_ref[b, row:row + 1, :] = _mm(oh, fs[g], _DN_1_0)
            if j == _KPS - 1:
                continue
            fcn = _mm(oh, fnorms[g], _DN_1_0)
            diff = fnorms[g] - fcn
            d2fn = _mm(ones_d, diff * diff, _DN_1_1)
            Ec = _mm(Es[g], oh, _DN_1_1)
            dE = 2.0 * Es[g] - 2.0 * Ec
            d2E = jnp.sum(dE * dE, axis=0, keepdims=True)
            dcur = jnp.sqrt(jnp.maximum(d2E + d2fn, 0.0))
            min_d[g] = dcur if min_d[g] is None else jnp.minimum(min_d[g], dcur)
            probs = min_d[g] * min_d[g]
            probs = probs / (jnp.sum(probs) + 1e-8)
            score = jnp.log(probs + 1e-20) + g_refs[_SCALES[si]][b, j:j + 1, :]
            mx = jnp.max(score)
            c_idx[g] = jnp.min(jnp.where(score == mx, il, n))


def kernel(features):
    X = features.reshape(_B, _HW * _HW, _D)
    full = lambda a: pl.BlockSpec(a.shape, lambda: tuple(0 for _ in a.shape))
    args = (X, _POOL[8], _POOL[16], _VINITS[0], _VINITS[1], _VINITS[2],
            _GUMBS[0], _GUMBS[1], _GUMBS[2])
    return pl.pallas_call(
        _mega_body,
        in_specs=[full(a) for a in args] + [pl.BlockSpec(memory_space=pltpu.SMEM)],
        out_specs=pl.BlockSpec((_B, len(_SCALES) * _KPS, _D), lambda: (0, 0, 0)),
        out_shape=jax.ShapeDtypeStruct((_B, len(_SCALES) * _KPS, _D), _F32),
    )(*args, _C0)


# speedup vs baseline: 28.5251x; 1.9631x over previous
import numpy as np
import jax
import jax.numpy as jnp
from jax import lax
from jax.experimental import pallas as pl
from jax.experimental.pallas import tpu as pltpu

_SCALES = (8, 16, 32)
_KPS = 4
_KNN = 20
_NPI = 50
_B = 2
_HW = 32
_D = 768
_NG = len(_SCALES) * _B


def _pool_matrix(s):
    bs = _HW // s
    P = np.zeros((s * s, _HW * _HW), np.float32)
    for i in range(s):
        for j in range(s):
            for y in range(i * bs, (i + 1) * bs):
                for x in range(j * bs, (j + 1) * bs):
                    P[i * s + j, y * _HW + x] = 1.0 / (bs * bs)
    return P


def _rng_consts():
    key = jax.random.key(42)
    vinits, gumbs, c0s = [], [], []
    for s in _SCALES:
        n = s * s
        vinit = np.zeros((_B, _KPS - 1, n), np.float32)
        gumb = np.zeros((_B, _KPS - 1, n), np.float32)
        for b in range(_B):
            for i in range(_KPS - 1):
                key, sk = jax.random.split(key)
                vinit[b, i] = np.asarray(jax.random.normal(sk, (n,), jnp.float32))
            key, sk = jax.random.split(key)
            c0s.append(int(jax.random.randint(sk, (), 0, n)))
            for j in range(_KPS - 1):
                key, sk = jax.random.split(key)
                gumb[b, j] = np.asarray(jax.random.gumbel(sk, (n,), jnp.float32))
        vinits.append(jnp.asarray(vinit))
        gumbs.append(jnp.asarray(gumb))
    return vinits, gumbs, jnp.asarray(np.asarray(c0s, np.int32))


_POOL = {s: _pool_matrix(s) for s in _SCALES if s != _HW}
_VINITS, _GUMBS, _C0 = _rng_consts()

_F32 = jnp.float32
_DN_1_1 = (((1,), (1,)), ((), ()))
_DN_1_0 = (((1,), (0,)), ((), ()))


def _mm(a, b, dn):
    return lax.dot_general(a, b, dn, preferred_element_type=_F32)


def _mega_body(x_ref, p8_ref, p16_ref, v8_ref, v16_ref, v32_ref,
               g8_ref, g16_ref, g32_ref, c0_ref, out_ref):
    ones_d = jnp.ones((1, _D), _F32)
    p_refs = {8: p8_ref, 16: p16_ref}
    v_refs = {8: v8_ref, 16: v16_ref, 32: v32_ref}
    g_refs = {8: g8_ref, 16: g16_ref, 32: g32_ref}

    fs, fn_cols, dmats = [], [], []
    for si, s in enumerate(_SCALES):
        n = s * s
        for b in range(_B):
            x = x_ref[b]
            f = x if s == _HW else _mm(p_refs[s][...], x, _DN_1_0)
            fsq = f * f
            fn_col = _mm(fsq, ones_d, _DN_1_1)
            fn_row = _mm(ones_d, fsq, _DN_1_1)
            G = _mm(f, f, _DN_1_1)
            dmat = jnp.sqrt(jnp.maximum(fn_col + fn_row - 2.0 * G, 0.0))
            fs.append(f)
            fn_cols.append(fn_col)
            dmats.append(dmat)

    big = _F32(np.inf)

    def thr_body(_, ms):
        return tuple(
            jnp.min(jnp.where(d > m, d, big), axis=1, keepdims=True)
            for d, m in zip(dmats, ms))

    thrs = lax.fori_loop(
        0, _KNN, thr_body,
        tuple(jnp.min(d, axis=1, keepdims=True) for d in dmats))

    Ms = []
    for g in range(_NG):
        n = dmats[g].shape[0]
        ri = lax.broadcasted_iota(jnp.int32, (n, n), 0)
        ci = lax.broadcasted_iota(jnp.int32, (n, n), 1)
        eye_b = ri == ci
        sel = (dmats[g] <= thrs[g]) & jnp.logical_not(eye_b)
        A = jnp.where(sel, jnp.exp(dmats[g] * -0.5), 0.0)
        eye_f = jnp.where(eye_b, _F32(1.0), _F32(0.0))
        At = _mm(A, eye_f, (((0,), (0,)), ((), ())))
        Wm = 0.5 * (A + At)
        ones_n = jnp.ones((1, n), _F32)
        Dv = _mm(ones_n, Wm, _DN_1_1)
        Dis = lax.rsqrt(Dv + 1e-8)
        Dis_col = _mm(Dis, eye_f, _DN_1_1)
        Ms.append(Wm * Dis * Dis_col)

    def norm1(v):
        return v / (jnp.sqrt(jnp.sum(v * v)) + 1e-8)

    evs = [[] for _ in range(_NG)]
    for i in range(_KPS):
        v0s, M2s = [], []
        for g in range(_NG):
            si, b = divmod(g, _B)
            if i == 0:
                v0 = _mm(ones_d, fs[g], _DN_1_1) * _F32(1.0 / _D)
                Mt = Ms[g]
            else:
                v0 = v_refs[_SCALES[si]][b, i - 1:i, :]
                E = jnp.concatenate(evs[g], axis=0)
                EM = _mm(E, Ms[g], _DN_1_0)
                Mt = Ms[g] - _mm(E, EM, (((0,), (0,)), ((), ())))
            v0s.append(norm1(v0))
            M2s.append(_mm(Mt, Mt, _DN_1_0))

        def pbody(_, vs, M2s_t=tuple(M2s)):
            out = []
            for g in range(_NG):
                vn = vs[g]
                for _u in range(5):
                    vn = _mm(vn, M2s_t[g], _DN_1_1)
                out.append(norm1(vn))
            return tuple(out)

        vs = lax.fori_loop(0, _NPI // 10, pbody, tuple(v0s))
        for g in range(_NG):
            evs[g].append(vs[g])

    fnorms, Es, ils, c_idx, min_d = [], [], [], [], [None] * _NG
    for g in range(_NG):
        n = fs[g].shape[0]
        fnorms.append(fs[g] / jnp.maximum(jnp.sqrt(fn_cols[g]), 1e-12))
        Es.append(jnp.concatenate(evs[g], axis=0))
        ils.append(lax.broadcasted_iota(jnp.int32, (1, n), 1))
        c_idx.append(c0_ref[g])

    for j in range(_KPS):
        for g in range(_NG):
            si, b = divmod(g, _B)
            n = fs[g].shape[0]
            il = ils[g]
            oh = jnp.where(il == c_idx[g], _F32(1.0), _F32(0.0))
            row = si * _KPS + j
            out_ref[b, row:row + 1, :] = _mm(oh, fs[g], _DN_1_0)
            if j == _KPS - 1:
                continue
            fcn = _mm(oh, fnorms[g], _DN_1_0)
            diff = fnorms[g] - fcn
            d2fn = _mm(ones_d, diff * diff, _DN_1_1)
            Ec = _mm(Es[g], oh, _DN_1_1)
            dE = 2.0 * Es[g] - 2.0 * Ec
            d2E = jnp.sum(dE * dE, axis=0, keepdims=True)
            dcur = jnp.sqrt(jnp.maximum(d2E + d2fn, 0.0))
            min_d[g] = dcur if min_d[g] is None else jnp.minimum(min_d[g], dcur)
            probs = min_d[g] * min_d[g]
            probs = probs / (jnp.sum(probs) + 1e-8)
            score = jnp.log(probs + 1e-20) + g_refs[_SCALES[si]][b, j:j + 1, :]
            mx = jnp.max(score)
            c_idx[g] = jnp.min(jnp.where(score == mx, il, n))


def kernel(features):
    X = features.reshape(_B, _HW * _HW, _D)
    full = lambda a: pl.BlockSpec(a.shape, lambda: tuple(0 for _ in a.shape))
    args = (X, _POOL[8], _POOL[16], _VINITS[0], _VINITS[1], _VINITS[2],
            _GUMBS[0], _GUMBS[1], _GUMBS[2])
    return pl.pallas_call(
        _mega_body,
        in_specs=[full(a) for a in args] + [pl.BlockSpec(memory_space=pltpu.SMEM)],
        out_specs=pl.BlockSpec((_B, len(_SCALES) * _KPS, _D), lambda: (0, 0, 0)),
        out_shape=jax.ShapeDtypeStruct((_B, len(_SCALES) * _KPS, _D), _F32),
    )(*args, _C0)
